# baseline (speedup 1.0000x reference)
import functools

import jax
import jax.numpy as jnp
from jax import lax
from jax.experimental import layout as jex_layout
from jax.experimental import pallas as pl
from jax.experimental.pallas import tpu as pltpu
from jax.experimental.pallas import tpu_sc as plsc

_L = 16
_NW = 32


def _sc_flip(B, K, C0, C1, H, W):
    N0 = B * K * C0
    N1 = B * K * C1

    mesh = plsc.VectorSubcoreMesh(core_axis_name="c", subcore_axis_name="s")

    @functools.partial(
        pl.kernel,
        mesh=mesh,
        out_type=[
            jax.ShapeDtypeStruct((N0, H, W), jnp.float32),
            jax.ShapeDtypeStruct((N1, H, W), jnp.float32),
        ],
        scratch_types=[
            pltpu.VMEM((_NW,), jnp.int32),
            pltpu.VMEM((2, H, W), jnp.float32),
            pltpu.VMEM((H, W), jnp.float32),
            pltpu.SemaphoreType.DMA,
            pltpu.SemaphoreType.DMA,
            pltpu.SemaphoreType.DMA,
        ],
    )
    def body(f0_hbm, f1_hbm, fi_hbm, o0_hbm, o1_hbm, fi_v, inb2, outb,
             sem_in0, sem_in1, sem_out):
        w = lax.axis_index("s") * 2 + lax.axis_index("c")
        pltpu.sync_copy(fi_hbm, fi_v)

        woffs = tuple(range(0, W - _L, _L)) + (W - _L,)

        def flip_plane(inref, negate):
            def row(h, carry):
                for woff in woffs:
                    v = inref[h, pl.ds(W - _L - woff, _L)]
                    v = lax.rev(v, (0,))
                    outb[h, pl.ds(woff, _L)] = -v if negate else v
                return carry
            lax.fori_loop(0, H, row, 0)

        in_sems = (sem_in0, sem_in1)

        def wait_out(o_hbm):
            pltpu.make_async_copy(outb, o_hbm.at[0], sem_out).wait()

        def wait_in(f_hbm, slot):
            pltpu.make_async_copy(f_hbm.at[0], inb2.at[slot], in_sems[slot]).wait()

        def src0(p):
            b = p // K
            k = p - b * K
            return b * K + fi_v[pl.ds(k, _L)][0]

        def src1(p):
            c = p % C1
            bk = p // C1
            b = bk // K
            k = bk - b * K
            return (b * K + fi_v[pl.ds(k, _L)][0]) * C1 + c

        def clamp0(i):
            return jnp.minimum(w + _NW * i, N0 - 1)

        def clamp1(i):
            return jnp.minimum(w + _NW * i, N1 - 1)

        NI0 = 2 * ((-(-N0 // _NW) + 1) // 2)
        NI1 = 2 * ((-(-N1 // _NW) + 1) // 2)

        pltpu.async_copy(f0_hbm.at[src0(clamp0(0))], inb2.at[0], sem_in0)

        def pair0(g, carry):
            for slot in (0, 1):
                i = 2 * g + slot
                p = clamp0(i)
                pltpu.async_copy(
                    f0_hbm.at[src0(clamp0(i + 1))], inb2.at[1 - slot],
                    in_sems[1 - slot])
                if slot == 0:
                    @pl.when(g > 0)
                    def _():
                        wait_out(o0_hbm)
                else:
                    wait_out(o0_hbm)
                wait_in(f0_hbm, slot)
                flip_plane(inb2.at[slot], False)
                pltpu.async_copy(outb, o0_hbm.at[p], sem_out)
            return carry

        lax.fori_loop(0, NI0 // 2, pair0, 0)
        wait_in(f0_hbm, 0)

        pltpu.async_copy(f1_hbm.at[src1(clamp1(0))], inb2.at[0], sem_in0)

        def pair1(g, carry):
            for slot in (0, 1):
                i = 2 * g + slot
                p = clamp1(i)
                c = p % C1
                pltpu.async_copy(
                    f1_hbm.at[src1(clamp1(i + 1))], inb2.at[1 - slot],
                    in_sems[1 - slot])
                if slot == 0:
                    @pl.when(g == 0)
                    def _():
                        wait_out(o0_hbm)

                    @pl.when(g > 0)
                    def _():
                        wait_out(o1_hbm)
                else:
                    wait_out(o1_hbm)
                wait_in(f1_hbm, slot)

                @pl.when(c == 0)
                def _():
                    flip_plane(inb2.at[slot], True)

                @pl.when(c != 0)
                def _():
                    flip_plane(inb2.at[slot], False)

                pltpu.async_copy(outb, o1_hbm.at[p], sem_out)
            return carry

        lax.fori_loop(0, NI1 // 2, pair1, 0)
        wait_in(f1_hbm, 0)
        wait_out(o1_hbm)

    return body


def kernel(field0, field1, flip_indices):
    B, K, C0, H, W = field0.shape
    C1 = field1.shape[2]
    lay5 = jex_layout.Layout(major_to_minor=(0, 1, 2, 3, 4))
    field0, field1 = lax.optimization_barrier((field0, field1))
    field0 = jex_layout.with_layout_constraint(field0, lay5)
    field1 = jex_layout.with_layout_constraint(field1, lay5)
    f0v = field0.reshape(B * K * C0, H, W)
    f1v = field1.reshape(B * K * C1, H, W)
    fi = jnp.pad(flip_indices.astype(jnp.int32), (0, _NW - flip_indices.shape[0]))

    o0, o1 = _sc_flip(B, K, C0, C1, H, W)(f0v, f1v, fi)
    return (o0.reshape(field0.shape), o1.reshape(field1.shape))


# speedup vs baseline: 1.0623x; 1.0623x over previous
import functools

import jax
import jax.numpy as jnp
from jax import lax
from jax.experimental import layout as jex_layout
from jax.experimental import pallas as pl
from jax.experimental.pallas import tpu as pltpu
from jax.experimental.pallas import tpu_sc as plsc

_L = 16
_NW = 32


def _sc_flip(B, K, C0, C1, H, W):
    N0 = B * K * C0
    N1 = B * K * C1

    mesh = plsc.VectorSubcoreMesh(core_axis_name="c", subcore_axis_name="s")

    @functools.partial(
        pl.kernel,
        mesh=mesh,
        out_type=[
            jax.ShapeDtypeStruct((N0, H, W), jnp.float32),
            jax.ShapeDtypeStruct((N1, H, W), jnp.float32),
        ],
        scratch_types=[
            pltpu.VMEM((_NW,), jnp.int32),
            pltpu.VMEM((H, W), jnp.float32),
            pltpu.VMEM((H, W), jnp.float32),
            pltpu.SemaphoreType.DMA,
            pltpu.SemaphoreType.DMA,
        ],
    )
    def body(f0_hbm, f1_hbm, fi_hbm, o0_hbm, o1_hbm, fi_v, inb, outb,
             sem_in, sem_out):
        w = lax.axis_index("s") * 2 + lax.axis_index("c")
        pltpu.sync_copy(fi_hbm, fi_v)

        woffs = tuple(range(0, W - _L, _L)) + (W - _L,)

        def flip_plane(negate):
            def row(h, carry):
                for woff in woffs:
                    v = inb[h, pl.ds(W - _L - woff, _L)]
                    v = lax.rev(v, (0,))
                    outb[h, pl.ds(woff, _L)] = -v if negate else v
                return carry
            lax.fori_loop(0, H, row, 0)

        def wait_out(o_hbm):
            pltpu.make_async_copy(outb, o_hbm.at[0], sem_out).wait()

        def do_field0(i, carry):
            p = jnp.minimum(w + _NW * i, N0 - 1)
            b = p // K
            k = p - b * K
            src = b * K + fi_v[pl.ds(k, _L)][0]
            in_dma = pltpu.async_copy(f0_hbm.at[src], inb, sem_in)

            @pl.when(i > 0)
            def _():
                wait_out(o0_hbm)

            in_dma.wait()
            flip_plane(False)
            pltpu.async_copy(outb, o0_hbm.at[p], sem_out)
            return carry

        lax.fori_loop(0, -(-N0 // _NW), do_field0, 0)

        def do_field1(i, carry):
            p = w + _NW * i
            c = p % C1
            bk = p // C1
            b = bk // K
            k = bk - b * K
            src = (b * K + fi_v[pl.ds(k, _L)][0]) * C1 + c
            in_dma = pltpu.async_copy(f1_hbm.at[src], inb, sem_in)

            @pl.when(i == 0)
            def _():
                wait_out(o0_hbm)

            @pl.when(i > 0)
            def _():
                wait_out(o1_hbm)

            in_dma.wait()

            @pl.when(c == 0)
            def _():
                flip_plane(True)

            @pl.when(c != 0)
            def _():
                flip_plane(False)

            pltpu.async_copy(outb, o1_hbm.at[p], sem_out)
            return carry

        lax.fori_loop(0, N1 // _NW, do_field1, 0)
        wait_out(o1_hbm)

    return body


def kernel(field0, field1, flip_indices):
    B, K, C0, H, W = field0.shape
    C1 = field1.shape[2]
    lay5 = jex_layout.Layout(major_to_minor=(0, 1, 2, 3, 4))
    field0, field1 = lax.optimization_barrier((field0, field1))
    field0 = jex_layout.with_layout_constraint(field0, lay5)
    field1 = jex_layout.with_layout_constraint(field1, lay5)
    f0v = field0.reshape(B * K * C0, H, W)
    f1v = field1.reshape(B * K * C1, H, W)
    fi = jnp.pad(flip_indices.astype(jnp.int32), (0, _NW - flip_indices.shape[0]))

    o0, o1 = _sc_flip(B, K, C0, C1, H, W)(f0v, f1v, fi)
    return (o0.reshape(field0.shape), o1.reshape(field1.shape))
